# baseline (speedup 1.0000x reference)
import functools

import jax
import jax.numpy as jnp
from jax import lax
from jax.experimental import pallas as pl
from jax.experimental.pallas import tpu as pltpu
from jax.experimental.pallas import tpu_sc as plsc

N_ROWS = 16384
IN_CH = 2048
L = 16
NC = 2
NS = 16
NW = NC * NS
ROWS_PER_W = N_ROWS // NW
CHUNK = 4
NG = IN_CH // L
UNROLL = 8
NBI = 2
NBO = 2
SLAB = 8
NSL = 3
CPS = SLAB // CHUNK
N_SLABS = ROWS_PER_W // SLAB
SH_ROWS = NS * NSL * SLAB


def _permute_body(x_hbm, p_hbm, out_hbm, p_v, *rest):
    xins = rest[:NBI]
    xouts = rest[NBI:NBI + NBO]
    sis = rest[NBI + NBO:2 * NBI + NBO]
    sos = rest[2 * NBI + NBO:2 * NBI + 2 * NBO]
    shared = rest[2 * NBI + 2 * NBO]
    ssl = rest[2 * NBI + 2 * NBO + 1:2 * NBI + 2 * NBO + 1 + NSL]

    cid = lax.axis_index("c")
    sid = lax.axis_index("s")
    wid = sid * NC + cid
    row0 = wid * ROWS_PER_W
    sh0 = sid * (NSL * SLAB)
    pltpu.sync_copy(p_hbm, p_v)

    def start_slab(s, sb):
        src = x_hbm.at[pl.ds(row0 + s * SLAB, SLAB), :]
        pltpu.async_copy(src, shared.at[pl.ds(sh0 + sb * SLAB, SLAB), :], ssl[sb])

    def wait_slab(sb):
        pltpu.make_async_copy(x_hbm.at[pl.ds(row0, SLAB), :],
                              shared.at[pl.ds(sh0, SLAB), :], ssl[sb]).wait()

    def start_in(sb, j, bi):
        src = shared.at[pl.ds(sh0 + sb * SLAB + j * CHUNK, CHUNK), :]
        pltpu.async_copy(src, xins[bi], sis[bi])

    def wait_in(bi):
        pltpu.make_async_copy(shared.at[pl.ds(sh0, CHUNK), :], xins[bi], sis[bi]).wait()

    def start_out(c, bo):
        dst = out_hbm.at[pl.ds(row0 + c * CHUNK, CHUNK), :]
        pltpu.async_copy(xouts[bo], dst, sos[bo])

    def wait_out(bo):
        pltpu.make_async_copy(xouts[bo], out_hbm.at[pl.ds(row0, CHUNK), :], sos[bo]).wait()

    for sb in range(NSL):
        start_slab(sb, sb)

    def do_chunk(c, bi, bo):
        @pl.when(c >= NBO)
        def _():
            wait_out(bo)
        wait_in(bi)

        @plsc.parallel_loop(0, NG, 1, unroll=UNROLL)
        def _(g):
            off = pl.multiple_of(g * L, L)
            idx = p_v[pl.ds(off, L)]
            for r in range(CHUNK):
                row_idx = jnp.full((L,), r, jnp.int32)
                v = plsc.load_gather(xins[bi], [row_idx, idx])
                xouts[bo][r, pl.ds(off, L)] = v

        start_out(c, bo)

    def slab_body(s, sb):
        wait_slab(sb)
        for j in range(CPS):
            start_in(sb, j, j % NBI)
        for j in range(CPS):
            do_chunk(s * CPS + j, j % NBI, (sb * CPS + j) % NBO)

        @pl.when(s + NSL < N_SLABS)
        def _():
            start_slab(s + NSL, sb)

    def ring_body(i, carry):
        for sb in range(NSL):
            slab_body(NSL * i + sb, sb)
        return carry

    lax.fori_loop(0, N_SLABS // NSL, ring_body, 0)
    for s in range(N_SLABS - (N_SLABS % NSL), N_SLABS):
        slab_body(s, s % NSL)
    for bo in range(NBO):
        wait_out(bo)


@jax.jit
def _permute(x, p):
    mesh = plsc.VectorSubcoreMesh(core_axis_name="c", subcore_axis_name="s")
    return pl.kernel(
        _permute_body,
        out_type=jax.ShapeDtypeStruct((N_ROWS, IN_CH), jnp.float32),
        mesh=mesh,
        scratch_types=(
            [pltpu.VMEM((IN_CH,), jnp.int32)]
            + [pltpu.VMEM((CHUNK, IN_CH), jnp.float32) for _ in range(NBI + NBO)]
            + [pltpu.SemaphoreType.DMA for _ in range(NBI + NBO)]
            + [pltpu.VMEM_SHARED((SH_ROWS, IN_CH), jnp.float32)]
            + [pltpu.SemaphoreType.DMA for _ in range(NSL)]
        ),
        compiler_params=pltpu.CompilerParams(needs_layout_passes=False),
    )(x, p)


def kernel(x, p):
    out = _permute(x, p.astype(jnp.int32))
    return (out, 0)


# speedup vs baseline: 1.2030x; 1.2030x over previous
import functools

import jax
import jax.numpy as jnp
from jax import lax
from jax.experimental import pallas as pl
from jax.experimental.pallas import tpu as pltpu
from jax.experimental.pallas import tpu_sc as plsc

N_ROWS = 16384
IN_CH = 2048
L = 16
NC = 2
NS = 16
NW = NC * NS
ROWS_PER_W = N_ROWS // NW
CHUNK = 8
CE = CHUNK * IN_CH
N_STEPS = ROWS_PER_W // CHUNK
NG = IN_CH // L
UNROLL = 8
NBUF = 3


def _permute_body(x_hbm, p_hbm, out_hbm, p_v, *rest):
    xins = rest[:NBUF]
    xouts = rest[NBUF:2 * NBUF]
    sis = rest[2 * NBUF:3 * NBUF]
    sos = rest[3 * NBUF:4 * NBUF]

    wid = lax.axis_index("s") * NC + lax.axis_index("c")
    row0 = wid * ROWS_PER_W
    pltpu.sync_copy(p_hbm, p_v)

    def start_in(c, b):
        src = x_hbm.at[pl.ds(row0 + c * CHUNK, CHUNK), :]
        pltpu.async_copy(src, xins[b], sis[b])

    def start_out(c, b):
        dst = out_hbm.at[pl.ds(row0 + c * CHUNK, CHUNK), :]
        pltpu.async_copy(xouts[b], dst, sos[b])

    def wait_in(b):
        pltpu.make_async_copy(x_hbm.at[pl.ds(row0, CHUNK), :], xins[b], sis[b]).wait()

    def wait_out(b):
        pltpu.make_async_copy(xouts[b], out_hbm.at[pl.ds(row0, CHUNK), :], sos[b]).wait()

    for b in range(NBUF):
        start_in(b, b)

    def chunk_body(c, b):
        @pl.when(c >= NBUF)
        def _():
            wait_out(b)
        wait_in(b)

        @plsc.parallel_loop(0, NG, 1, unroll=UNROLL)
        def _(g):
            off = pl.multiple_of(g * L, L)
            idx = p_v[pl.ds(off, L)]
            for r in range(CHUNK):
                row_idx = jnp.full((L,), r, jnp.int32)
                v = plsc.load_gather(xins[b], [row_idx, idx])
                xouts[b][r, pl.ds(off, L)] = v

        start_out(c, b)

        @pl.when(c + NBUF < N_STEPS)
        def _():
            start_in(c + NBUF, b)

    def ring_body(i, carry):
        for b in range(NBUF):
            chunk_body(NBUF * i + b, b)
        return carry

    lax.fori_loop(0, N_STEPS // NBUF, ring_body, 0)
    for c in range(N_STEPS - (N_STEPS % NBUF), N_STEPS):
        chunk_body(c, c % NBUF)
    for b in range(NBUF):
        wait_out(b)


@jax.jit
def _permute(x, p):
    mesh = plsc.VectorSubcoreMesh(core_axis_name="c", subcore_axis_name="s")
    return pl.kernel(
        _permute_body,
        out_type=jax.ShapeDtypeStruct((N_ROWS, IN_CH), jnp.float32),
        mesh=mesh,
        scratch_types=(
            [pltpu.VMEM((IN_CH,), jnp.int32)]
            + [pltpu.VMEM((CHUNK, IN_CH), jnp.float32) for _ in range(2 * NBUF)]
            + [pltpu.SemaphoreType.DMA for _ in range(2 * NBUF)]
        ),
        compiler_params=pltpu.CompilerParams(needs_layout_passes=False),
    )(x, p)


def kernel(x, p):
    out = _permute(x, p.astype(jnp.int32))
    return (out, 0)
